# baseline (speedup 1.0000x reference)
import functools

import jax
import jax.numpy as jnp
from jax import lax
from jax.experimental import pallas as pl
from jax.experimental.pallas import tpu as pltpu
from jax.experimental.pallas import tpu_sc as plsc

N = 10000
D = 128
E = 160000
NTILE = 16
CH = 84
GN = CH // 4
EP = NTILE * CH * 128
CI = 5
NP = NTILE * CI * 128
ACC_ROWS = N + 8
RPT = 624
WB = 104
TAIL = N - NTILE * RPT

_mesh = plsc.VectorSubcoreMesh(core_axis_name="c", subcore_axis_name="s")
_f32 = jnp.float32
_i32 = jnp.int32


def _make_jobs(tile, s_v, d_v, rows, acc, gsems, ssems):
  r0 = rows[0]

  def init_job(table, fidx, outp):
    def body(j, carry):
      pltpu.sync_copy(fidx.at[tile, j], s_v.at[0])
      pltpu.async_copy(table.at[s_v.at[0]], r0, gsems[0]).wait()
      pltpu.sync_copy(r0, outp.at[pl.ds(tile * CI * 128 + j * 128, 128)])
      return carry

    lax.fori_loop(0, CI, body, 0)

  def seg_job(table, s2, d2, outp, base):
    def ib(k, carry):
      off = tile * RPT + k * WB
      pltpu.sync_copy(base.at[pl.ds(off, WB)], r0.at[pl.ds(0, WB)])
      pltpu.sync_copy(r0.at[pl.ds(0, WB)], acc.at[pl.ds(off, WB)])
      return carry

    lax.fori_loop(0, RPT // WB, ib, 0)

    @pl.when(tile == 0)
    def _():
      pltpu.sync_copy(base.at[pl.ds(NTILE * RPT, TAIL)], r0.at[pl.ds(0, TAIL)])
      pltpu.sync_copy(r0.at[pl.ds(0, TAIL)], acc.at[pl.ds(NTILE * RPT, TAIL)])

    plsc.subcore_barrier()

    def grp(q, carry):
      @pl.when(q > 0)
      def _():
        for b in range(2):
          pltpu.make_async_copy(rows[b], acc.at[d_v.at[b + 2]],
                                ssems[b]).wait()
      pltpu.sync_copy(s2.at[tile, q], s_v)
      pltpu.sync_copy(d2.at[tile, q], d_v)
      cg = [pltpu.async_copy(table.at[s_v.at[b]], rows[b], gsems[b])
            for b in range(2)]
      cs = []
      for b in range(2):
        cg[b].wait()
        cs.append(
            pltpu.async_copy(rows[b], acc.at[d_v.at[b]], ssems[b], add=True))
      cg2 = []
      for b in range(2):
        cs[b].wait()
        cg2.append(
            pltpu.async_copy(table.at[s_v.at[b + 2]], rows[b], gsems[b]))
      for b in range(2):
        cg2[b].wait()
        pltpu.async_copy(rows[b], acc.at[d_v.at[b + 2]], ssems[b], add=True)
      return carry

    lax.fori_loop(0, GN, grp, 0)
    for b in range(2):
      pltpu.make_async_copy(rows[b], acc.at[d_v.at[b + 2]], ssems[b]).wait()
    plsc.subcore_barrier()

    def ob(k, carry):
      off = tile * RPT + k * WB
      pltpu.sync_copy(acc.at[pl.ds(off, WB)], r0.at[pl.ds(0, WB)])
      pltpu.sync_copy(r0.at[pl.ds(0, WB)], outp.at[pl.ds(off, WB)])
      return carry

    lax.fori_loop(0, RPT // WB, ob, 0)

    @pl.when(tile == 0)
    def _():
      pltpu.sync_copy(acc.at[pl.ds(NTILE * RPT, TAIL)], rows[1].at[pl.ds(0, TAIL)])
      pltpu.sync_copy(rows[1].at[pl.ds(0, TAIL)],
                      outp.at[pl.ds(NTILE * RPT, TAIL)])

  return init_job, seg_job


def _pre_body(t360, t18, zz, f0i, f1i, fa101, dd101, fa021, dd021, fa110,
              dd110, fa030, dd030, h0p, h1p, p101, p021, p110, p030,
              s_v, d_v, r0, r1, acc, gs0, gs1, ss0, ss1):
  tile = lax.axis_index("s")
  core = lax.axis_index("c")
  init_job, seg_job = _make_jobs(tile, s_v, d_v, (r0, r1), acc,
                                 (gs0, gs1), (ss0, ss1))

  @pl.when(core == 0)
  def _():
    init_job(t360, f0i, h0p)
    seg_job(t18, fa101, dd101, p101, zz)
    seg_job(t18, fa030, dd030, p030, zz)

  @pl.when(core == 1)
  def _():
    init_job(t360, f1i, h1p)
    seg_job(t18, fa021, dd021, p021, zz)
    seg_job(t18, fa110, dd110, p110, zz)


def _layer_body(h0, h1, q101, q021, q110, q030, s101, dd101, s021, dd021,
                s110, dd110, s030, dd030, a101, a021, a110, a030,
                s_v, d_v, r0, r1, acc, gs0, gs1, ss0, ss1):
  tile = lax.axis_index("s")
  core = lax.axis_index("c")
  _, seg_job = _make_jobs(tile, s_v, d_v, (r0, r1), acc,
                          (gs0, gs1), (ss0, ss1))

  @pl.when(core == 0)
  def _():
    seg_job(h1, s101, dd101, a101, q101)
    seg_job(h0, s030, dd030, a030, q030)

  @pl.when(core == 1)
  def _():
    seg_job(h0, s021, dd021, a021, q021)
    seg_job(h1, s110, dd110, a110, q110)


_SC_SCRATCH = [
    pltpu.VMEM((4, 128), _i32),
    pltpu.VMEM((4, 128), _i32),
    pltpu.VMEM((128, D), _f32),
    pltpu.VMEM((128, D), _f32),
    pltpu.VMEM_SHARED((ACC_ROWS, D), _f32),
    pltpu.SemaphoreType.DMA,
    pltpu.SemaphoreType.DMA,
    pltpu.SemaphoreType.DMA,
    pltpu.SemaphoreType.DMA,
]

_sc_pre = pl.kernel(
    _pre_body,
    out_type=[jax.ShapeDtypeStruct((NP, D), _f32),
              jax.ShapeDtypeStruct((NP, D), _f32)] +
             [jax.ShapeDtypeStruct((N, D), _f32)] * 4,
    mesh=_mesh,
    scratch_types=_SC_SCRATCH,
)

_sc_layer = pl.kernel(
    _layer_body,
    out_type=[jax.ShapeDtypeStruct((N, D), _f32)] * 4,
    mesh=_mesh,
    scratch_types=_SC_SCRATCH,
)


def _tc1_body(h1, a101, a021, gw1, gb1, gw2, gb2, w021, vb021, gam, bet, out,
              *, relu):
  x = a101[...] + 1.1 * h1[...]
  hh = jnp.maximum(
      jnp.dot(x, gw1[...], preferred_element_type=_f32) + gb1[...], 0.0)
  hgin = jnp.dot(hh, gw2[...], preferred_element_type=_f32) + gb2[...]
  o021 = (jnp.dot(a021[...], w021[...], preferred_element_type=_f32)
          + vb021[...]) * 0.1
  y = (hgin + o021) * 0.5
  m = jnp.mean(y, axis=0, keepdims=True)
  v = jnp.mean((y - m) ** 2, axis=0, keepdims=True)
  y = gam[...] * (y - m) * lax.rsqrt(v + 1e-5) + bet[...]
  if relu:
    y = jnp.maximum(y, 0.0)
  out[...] = y


def _tc0_body(a110, a030, w110, vb110, w030, vb030, gam, bet, out, *, relu):
  o110 = (jnp.dot(a110[...], w110[...], preferred_element_type=_f32)
          + vb110[...]) * 0.1
  o030 = (jnp.dot(a030[...], w030[...], preferred_element_type=_f32)
          + vb030[...]) * 0.1
  y = (o110 + o030) * 0.5
  m = jnp.mean(y, axis=0, keepdims=True)
  v = jnp.mean((y - m) ** 2, axis=0, keepdims=True)
  y = gam[...] * (y - m) * lax.rsqrt(v + 1e-5) + bet[...]
  if relu:
    y = jnp.maximum(y, 0.0)
  out[...] = y


def _tc1(relu):
  return pl.pallas_call(
      functools.partial(_tc1_body, relu=relu),
      out_shape=jax.ShapeDtypeStruct((N, D), _f32))


def _tc0(relu):
  return pl.pallas_call(
      functools.partial(_tc0_body, relu=relu),
      out_shape=jax.ShapeDtypeStruct((N, D), _f32))


def _pad_src(v, total, mod, shape=(NTILE, GN, 4, 128)):
  pad = total - v.shape[0]
  fill = jnp.arange(pad, dtype=_i32) % mod
  return jnp.concatenate([v.astype(_i32), fill]).reshape(shape)


def _pad_dst(v, total):
  pad = total - v.shape[0]
  fill = N + (jnp.arange(pad, dtype=_i32) % 8)
  return jnp.concatenate([v.astype(_i32), fill]).reshape(NTILE, GN, 4, 128)


def kernel(x0, x1, edge_index_101, edge_attr_101, edge_index_110,
           edge_attr_110, edge_index_021, edge_attr_021, edge_index_030,
           edge_attr_030, x_emb1, x_emb2, edge_emb1, edge_emb2, gin_W1,
           gin_b1, gin_W2, gin_b2, W110, b110, W021, b021, W030, b030,
           bn_gamma, bn_beta):
  R360, R18 = 28, 512
  t360 = jnp.tile((x_emb1[:, None, :] + x_emb2[None, :, :]).reshape(360, D),
                  (R360, 1))
  t18 = jnp.tile((edge_emb1[:, None, :] + edge_emb2[None, :, :]).reshape(18, D),
                 (R18, 1))
  zz = jnp.zeros((N, D), _f32)

  def spread(f, nrows, nrep):
    return f + nrows * (jnp.arange(f.shape[0], dtype=_i32) % nrep)

  f0i = _pad_src(spread(x0[:, 0] * 3 + x0[:, 1], 360, R360), NP, 360 * R360,
                 (NTILE, CI, 128))
  f1i = _pad_src(spread(x1[:, 0] * 3 + x1[:, 1], 360, R360), NP, 360 * R360,
                 (NTILE, CI, 128))

  def eidx(ei, ea):
    fa = _pad_src(spread(ea[:, 0] * 3 + ea[:, 1], 18, R18), EP, 18 * R18)
    s = _pad_src(ei[0], EP, N)
    dd = _pad_dst(ei[1], EP)
    return fa, s, dd

  fa101, s101, dd101 = eidx(edge_index_101, edge_attr_101)
  fa021, s021, dd021 = eidx(edge_index_021, edge_attr_021)
  fa110, s110, dd110 = eidx(edge_index_110, edge_attr_110)
  fa030, s030, dd030 = eidx(edge_index_030, edge_attr_030)

  h0p, h1p, p101, p021, p110, p030 = _sc_pre(
      t360, t18, zz, f0i, f1i, fa101, dd101, fa021, dd021, fa110, dd110,
      fa030, dd030)
  h0 = h0p[:N]
  h1 = h1p[:N]

  gb1 = gin_b1.reshape(1, -1)
  gb2 = gin_b2.reshape(1, -1)
  vb110 = b110.reshape(1, -1)
  vb021 = b021.reshape(1, -1)
  vb030 = b030.reshape(1, -1)

  for layer in range(2):
    a101, a021, a110, a030 = _sc_layer(
        h0, h1, p101, p021, p110, p030, s101, dd101, s021, dd021,
        s110, dd110, s030, dd030)
    gam = bn_gamma[layer].reshape(1, D)
    bet = bn_beta[layer].reshape(1, D)
    relu = layer == 0
    h1 = _tc1(relu)(h1, a101, a021, gin_W1, gb1, gin_W2, gb2, W021, vb021,
                    gam, bet)
    h0 = _tc0(relu)(a110, a030, W110, vb110, W030, vb030, gam, bet)

  return jnp.concatenate([h0, h1], axis=0)


# speedup vs baseline: 1.1204x; 1.1204x over previous
import functools

import jax
import jax.numpy as jnp
from jax import lax
from jax.experimental import pallas as pl
from jax.experimental.pallas import tpu as pltpu
from jax.experimental.pallas import tpu_sc as plsc

N = 10000
D = 128
E = 160000
NTILE = 16
CH = 80
GCH = 16
GN = CH // GCH
EP = NTILE * CH * 128
CI = 5
NP = NTILE * CI * 128
ACC_ROWS = N + 8
RPT = 624
WB = 104
TAIL = N - NTILE * RPT

_mesh = plsc.VectorSubcoreMesh(core_axis_name="c", subcore_axis_name="s")
_f32 = jnp.float32
_i32 = jnp.int32


def _make_jobs(tile, s_v, d_v, rows, acc, gsems, ssems):
  r0 = rows[0]

  def init_job(table, fidx, outp):
    def body(j, carry):
      pltpu.sync_copy(fidx.at[tile, j], s_v.at[0])
      pltpu.async_copy(table.at[s_v.at[0]], r0, gsems[0]).wait()
      pltpu.sync_copy(r0, outp.at[pl.ds(tile * CI * 128 + j * 128, 128)])
      return carry

    lax.fori_loop(0, CI, body, 0)

  def seg_job(table, s2, d2, outp, base):
    def ib(k, carry):
      off = tile * RPT + k * WB
      pltpu.sync_copy(base.at[pl.ds(off, WB)], r0.at[pl.ds(0, WB)])
      pltpu.sync_copy(r0.at[pl.ds(0, WB)], acc.at[pl.ds(off, WB)])
      return carry

    lax.fori_loop(0, RPT // WB, ib, 0)

    @pl.when(tile == 0)
    def _():
      pltpu.sync_copy(base.at[pl.ds(NTILE * RPT, TAIL)], r0.at[pl.ds(0, TAIL)])
      pltpu.sync_copy(r0.at[pl.ds(0, TAIL)], acc.at[pl.ds(NTILE * RPT, TAIL)])

    plsc.subcore_barrier()

    r1 = rows[1]

    def grp(q, carry):
      pltpu.sync_copy(s2.at[tile, pl.ds(q * GCH, GCH)], s_v)
      pltpu.sync_copy(d2.at[tile, pl.ds(q * GCH, GCH)], d_v)

      def eb(p, c2):
        j = 2 * p
        c0 = pltpu.async_copy(table.at[s_v.at[j]], r0, gsems[0])
        c1 = pltpu.async_copy(table.at[s_v.at[j + 1]], r1, gsems[1])
        c0.wait()
        pltpu.sync_copy(r0, acc.at[d_v.at[j]], add=True)
        c1.wait()
        pltpu.sync_copy(r1, acc.at[d_v.at[j + 1]], add=True)
        return c2

      lax.fori_loop(0, GCH // 2, eb, 0)
      return carry

    lax.fori_loop(0, GN, grp, 0)
    plsc.subcore_barrier()

    def ob(k, carry):
      off = tile * RPT + k * WB
      pltpu.sync_copy(acc.at[pl.ds(off, WB)], r0.at[pl.ds(0, WB)])
      pltpu.sync_copy(r0.at[pl.ds(0, WB)], outp.at[pl.ds(off, WB)])
      return carry

    lax.fori_loop(0, RPT // WB, ob, 0)

    @pl.when(tile == 0)
    def _():
      pltpu.sync_copy(acc.at[pl.ds(NTILE * RPT, TAIL)], rows[1].at[pl.ds(0, TAIL)])
      pltpu.sync_copy(rows[1].at[pl.ds(0, TAIL)],
                      outp.at[pl.ds(NTILE * RPT, TAIL)])

  return init_job, seg_job


def _pre_body(t360, t18, zz, f0i, f1i, fa101, dd101, fa021, dd021, fa110,
              dd110, fa030, dd030, h0p, h1p, p101, p021, p110, p030,
              s_v, d_v, r0, r1, acc, gs0, gs1, ss0, ss1):
  tile = lax.axis_index("s")
  core = lax.axis_index("c")
  init_job, seg_job = _make_jobs(tile, s_v, d_v, (r0, r1), acc,
                                 (gs0, gs1), (ss0, ss1))

  @pl.when(core == 0)
  def _():
    init_job(t360, f0i, h0p)
    seg_job(t18, fa101, dd101, p101, zz)
    seg_job(t18, fa030, dd030, p030, zz)

  @pl.when(core == 1)
  def _():
    init_job(t360, f1i, h1p)
    seg_job(t18, fa021, dd021, p021, zz)
    seg_job(t18, fa110, dd110, p110, zz)


def _layer_body(h0, h1, q101, q021, q110, q030, s101, dd101, s021, dd021,
                s110, dd110, s030, dd030, a101, a021, a110, a030,
                s_v, d_v, r0, r1, acc, gs0, gs1, ss0, ss1):
  tile = lax.axis_index("s")
  core = lax.axis_index("c")
  _, seg_job = _make_jobs(tile, s_v, d_v, (r0, r1), acc,
                          (gs0, gs1), (ss0, ss1))

  @pl.when(core == 0)
  def _():
    seg_job(h1, s101, dd101, a101, q101)
    seg_job(h0, s030, dd030, a030, q030)

  @pl.when(core == 1)
  def _():
    seg_job(h0, s021, dd021, a021, q021)
    seg_job(h1, s110, dd110, a110, q110)


_SC_SCRATCH = [
    pltpu.VMEM((GCH, 128), _i32),
    pltpu.VMEM((GCH, 128), _i32),
    pltpu.VMEM((128, D), _f32),
    pltpu.VMEM((128, D), _f32),
    pltpu.VMEM_SHARED((ACC_ROWS, D), _f32),
    pltpu.SemaphoreType.DMA,
    pltpu.SemaphoreType.DMA,
    pltpu.SemaphoreType.DMA,
    pltpu.SemaphoreType.DMA,
]

_sc_pre = pl.kernel(
    _pre_body,
    out_type=[jax.ShapeDtypeStruct((NP, D), _f32),
              jax.ShapeDtypeStruct((NP, D), _f32)] +
             [jax.ShapeDtypeStruct((N, D), _f32)] * 4,
    mesh=_mesh,
    scratch_types=_SC_SCRATCH,
)

_sc_layer = pl.kernel(
    _layer_body,
    out_type=[jax.ShapeDtypeStruct((N, D), _f32)] * 4,
    mesh=_mesh,
    scratch_types=_SC_SCRATCH,
)


def _tc1_body(h1, a101, a021, gw1, gb1, gw2, gb2, w021, vb021, gam, bet, out,
              *, relu):
  x = a101[...] + 1.1 * h1[...]
  hh = jnp.maximum(
      jnp.dot(x, gw1[...], preferred_element_type=_f32) + gb1[...], 0.0)
  hgin = jnp.dot(hh, gw2[...], preferred_element_type=_f32) + gb2[...]
  o021 = (jnp.dot(a021[...], w021[...], preferred_element_type=_f32)
          + vb021[...]) * 0.1
  y = (hgin + o021) * 0.5
  m = jnp.mean(y, axis=0, keepdims=True)
  v = jnp.mean((y - m) ** 2, axis=0, keepdims=True)
  y = gam[...] * (y - m) * lax.rsqrt(v + 1e-5) + bet[...]
  if relu:
    y = jnp.maximum(y, 0.0)
  out[...] = y


def _tc0_body(a110, a030, w110, vb110, w030, vb030, gam, bet, out, *, relu):
  o110 = (jnp.dot(a110[...], w110[...], preferred_element_type=_f32)
          + vb110[...]) * 0.1
  o030 = (jnp.dot(a030[...], w030[...], preferred_element_type=_f32)
          + vb030[...]) * 0.1
  y = (o110 + o030) * 0.5
  m = jnp.mean(y, axis=0, keepdims=True)
  v = jnp.mean((y - m) ** 2, axis=0, keepdims=True)
  y = gam[...] * (y - m) * lax.rsqrt(v + 1e-5) + bet[...]
  if relu:
    y = jnp.maximum(y, 0.0)
  out[...] = y


def _tc1(relu):
  return pl.pallas_call(
      functools.partial(_tc1_body, relu=relu),
      out_shape=jax.ShapeDtypeStruct((N, D), _f32))


def _tc0(relu):
  return pl.pallas_call(
      functools.partial(_tc0_body, relu=relu),
      out_shape=jax.ShapeDtypeStruct((N, D), _f32))


def _pad_src(v, total, mod, shape=(NTILE, CH, 128)):
  pad = total - v.shape[0]
  fill = jnp.arange(pad, dtype=_i32) % mod
  return jnp.concatenate([v.astype(_i32), fill]).reshape(shape)


def _pad_dst(v, total):
  pad = total - v.shape[0]
  fill = N + (jnp.arange(pad, dtype=_i32) % 8)
  return jnp.concatenate([v.astype(_i32), fill]).reshape(NTILE, CH, 128)


def kernel(x0, x1, edge_index_101, edge_attr_101, edge_index_110,
           edge_attr_110, edge_index_021, edge_attr_021, edge_index_030,
           edge_attr_030, x_emb1, x_emb2, edge_emb1, edge_emb2, gin_W1,
           gin_b1, gin_W2, gin_b2, W110, b110, W021, b021, W030, b030,
           bn_gamma, bn_beta):
  R360, R18 = 28, 512
  t360 = jnp.tile((x_emb1[:, None, :] + x_emb2[None, :, :]).reshape(360, D),
                  (R360, 1))
  t18 = jnp.tile((edge_emb1[:, None, :] + edge_emb2[None, :, :]).reshape(18, D),
                 (R18, 1))
  zz = jnp.zeros((N, D), _f32)

  def spread(f, nrows, nrep):
    return f + nrows * (jnp.arange(f.shape[0], dtype=_i32) % nrep)

  f0i = _pad_src(spread(x0[:, 0] * 3 + x0[:, 1], 360, R360), NP, 360 * R360,
                 (NTILE, CI, 128))
  f1i = _pad_src(spread(x1[:, 0] * 3 + x1[:, 1], 360, R360), NP, 360 * R360,
                 (NTILE, CI, 128))

  def eidx(ei, ea):
    fa = _pad_src(spread(ea[:, 0] * 3 + ea[:, 1], 18, R18), EP, 18 * R18)
    s = _pad_src(ei[0], EP, N)
    dd = _pad_dst(ei[1], EP)
    return fa, s, dd

  fa101, s101, dd101 = eidx(edge_index_101, edge_attr_101)
  fa021, s021, dd021 = eidx(edge_index_021, edge_attr_021)
  fa110, s110, dd110 = eidx(edge_index_110, edge_attr_110)
  fa030, s030, dd030 = eidx(edge_index_030, edge_attr_030)

  h0p, h1p, p101, p021, p110, p030 = _sc_pre(
      t360, t18, zz, f0i, f1i, fa101, dd101, fa021, dd021, fa110, dd110,
      fa030, dd030)
  h0 = h0p[:N]
  h1 = h1p[:N]

  gb1 = gin_b1.reshape(1, -1)
  gb2 = gin_b2.reshape(1, -1)
  vb110 = b110.reshape(1, -1)
  vb021 = b021.reshape(1, -1)
  vb030 = b030.reshape(1, -1)

  for layer in range(2):
    a101, a021, a110, a030 = _sc_layer(
        h0, h1, p101, p021, p110, p030, s101, dd101, s021, dd021,
        s110, dd110, s030, dd030)
    gam = bn_gamma[layer].reshape(1, D)
    bet = bn_beta[layer].reshape(1, D)
    relu = layer == 0
    h1 = _tc1(relu)(h1, a101, a021, gin_W1, gb1, gin_W2, gb2, W021, vb021,
                    gam, bet)
    h0 = _tc0(relu)(a110, a030, W110, vb110, W030, vb030, gam, bet)

  return jnp.concatenate([h0, h1], axis=0)


# speedup vs baseline: 1.1230x; 1.0023x over previous
import functools

import jax
import jax.numpy as jnp
from jax import lax
from jax.experimental import pallas as pl
from jax.experimental.pallas import tpu as pltpu
from jax.experimental.pallas import tpu_sc as plsc

N = 10000
D = 128
E = 160000
NTILE = 16
CH = 80
GCH = 16
GN = CH // GCH
EP = NTILE * CH * 128
CI = 5
NP = NTILE * CI * 128
ACC_ROWS = N + 8
RPT = 624
WB = 104
TAIL = N - NTILE * RPT

_mesh = plsc.VectorSubcoreMesh(core_axis_name="c", subcore_axis_name="s")
_f32 = jnp.float32
_i32 = jnp.int32


def _make_jobs(tile, s_v, d_v, rows, acc, gsems, ssems):
  r0 = rows[0]

  def init_job(table, fidx, outp):
    def body(j, carry):
      pltpu.sync_copy(fidx.at[tile, j], s_v.at[0])
      pltpu.async_copy(table.at[s_v.at[0]], r0, gsems[0]).wait()
      pltpu.sync_copy(r0, outp.at[pl.ds(tile * CI * 128 + j * 128, 128)])
      return carry

    lax.fori_loop(0, CI, body, 0)

  def seg_job(table, s2, d2, outp, base):
    def ib(k, carry):
      off = tile * RPT + k * WB
      pltpu.sync_copy(base.at[pl.ds(off, WB)], r0.at[pl.ds(0, WB)])
      pltpu.sync_copy(r0.at[pl.ds(0, WB)], acc.at[pl.ds(off, WB)])
      return carry

    lax.fori_loop(0, RPT // WB, ib, 0)

    @pl.when(tile == 0)
    def _():
      pltpu.sync_copy(base.at[pl.ds(NTILE * RPT, TAIL)], r0.at[pl.ds(0, TAIL)])
      pltpu.sync_copy(r0.at[pl.ds(0, TAIL)], acc.at[pl.ds(NTILE * RPT, TAIL)])

    plsc.subcore_barrier()

    r1 = rows[1]

    def eb(p, carry):
      @pl.when(p > 0)
      def _():
        pltpu.make_async_copy(r1, acc.at[d_v.at[15]], ssems[1]).wait()

      @pl.when(p % 8 == 0)
      def _():
        q = p // 8
        pltpu.sync_copy(s2.at[tile, pl.ds(q * GCH, GCH)], s_v)
        pltpu.sync_copy(d2.at[tile, pl.ds(q * GCH, GCH)], d_v)

      j = (p % 8) * 2
      c0 = pltpu.async_copy(table.at[s_v.at[j]], r0, gsems[0])
      c1 = pltpu.async_copy(table.at[s_v.at[j + 1]], r1, gsems[1])
      c0.wait()
      pltpu.sync_copy(r0, acc.at[d_v.at[j]], add=True)
      c1.wait()
      pltpu.async_copy(r1, acc.at[d_v.at[j + 1]], ssems[1], add=True)
      return carry

    lax.fori_loop(0, CH // 2, eb, 0)
    pltpu.make_async_copy(r1, acc.at[d_v.at[15]], ssems[1]).wait()
    plsc.subcore_barrier()

    def ob(k, carry):
      off = tile * RPT + k * WB
      pltpu.sync_copy(acc.at[pl.ds(off, WB)], r0.at[pl.ds(0, WB)])
      pltpu.sync_copy(r0.at[pl.ds(0, WB)], outp.at[pl.ds(off, WB)])
      return carry

    lax.fori_loop(0, RPT // WB, ob, 0)

    @pl.when(tile == 0)
    def _():
      pltpu.sync_copy(acc.at[pl.ds(NTILE * RPT, TAIL)], rows[1].at[pl.ds(0, TAIL)])
      pltpu.sync_copy(rows[1].at[pl.ds(0, TAIL)],
                      outp.at[pl.ds(NTILE * RPT, TAIL)])

  return init_job, seg_job


def _pre_body(t360, t18, zz, f0i, f1i, fa101, dd101, fa021, dd021, fa110,
              dd110, fa030, dd030, h0p, h1p, p101, p021, p110, p030,
              s_v, d_v, r0, r1, acc, gs0, gs1, ss0, ss1):
  tile = lax.axis_index("s")
  core = lax.axis_index("c")
  init_job, seg_job = _make_jobs(tile, s_v, d_v, (r0, r1), acc,
                                 (gs0, gs1), (ss0, ss1))

  @pl.when(core == 0)
  def _():
    init_job(t360, f0i, h0p)
    seg_job(t18, fa101, dd101, p101, zz)
    seg_job(t18, fa030, dd030, p030, zz)

  @pl.when(core == 1)
  def _():
    init_job(t360, f1i, h1p)
    seg_job(t18, fa021, dd021, p021, zz)
    seg_job(t18, fa110, dd110, p110, zz)


def _layer_body(h0, h1, q101, q021, q110, q030, s101, dd101, s021, dd021,
                s110, dd110, s030, dd030, a101, a021, a110, a030,
                s_v, d_v, r0, r1, acc, gs0, gs1, ss0, ss1):
  tile = lax.axis_index("s")
  core = lax.axis_index("c")
  _, seg_job = _make_jobs(tile, s_v, d_v, (r0, r1), acc,
                          (gs0, gs1), (ss0, ss1))

  @pl.when(core == 0)
  def _():
    seg_job(h1, s101, dd101, a101, q101)
    seg_job(h0, s030, dd030, a030, q030)

  @pl.when(core == 1)
  def _():
    seg_job(h0, s021, dd021, a021, q021)
    seg_job(h1, s110, dd110, a110, q110)


_SC_SCRATCH = [
    pltpu.VMEM((GCH, 128), _i32),
    pltpu.VMEM((GCH, 128), _i32),
    pltpu.VMEM((128, D), _f32),
    pltpu.VMEM((128, D), _f32),
    pltpu.VMEM_SHARED((ACC_ROWS, D), _f32),
    pltpu.SemaphoreType.DMA,
    pltpu.SemaphoreType.DMA,
    pltpu.SemaphoreType.DMA,
    pltpu.SemaphoreType.DMA,
]

_sc_pre = pl.kernel(
    _pre_body,
    out_type=[jax.ShapeDtypeStruct((NP, D), _f32),
              jax.ShapeDtypeStruct((NP, D), _f32)] +
             [jax.ShapeDtypeStruct((N, D), _f32)] * 4,
    mesh=_mesh,
    scratch_types=_SC_SCRATCH,
)

_sc_layer = pl.kernel(
    _layer_body,
    out_type=[jax.ShapeDtypeStruct((N, D), _f32)] * 4,
    mesh=_mesh,
    scratch_types=_SC_SCRATCH,
)


def _tc1_body(h1, a101, a021, gw1, gb1, gw2, gb2, w021, vb021, gam, bet, out,
              *, relu):
  x = a101[...] + 1.1 * h1[...]
  hh = jnp.maximum(
      jnp.dot(x, gw1[...], preferred_element_type=_f32) + gb1[...], 0.0)
  hgin = jnp.dot(hh, gw2[...], preferred_element_type=_f32) + gb2[...]
  o021 = (jnp.dot(a021[...], w021[...], preferred_element_type=_f32)
          + vb021[...]) * 0.1
  y = (hgin + o021) * 0.5
  m = jnp.mean(y, axis=0, keepdims=True)
  v = jnp.mean((y - m) ** 2, axis=0, keepdims=True)
  y = gam[...] * (y - m) * lax.rsqrt(v + 1e-5) + bet[...]
  if relu:
    y = jnp.maximum(y, 0.0)
  out[...] = y


def _tc0_body(a110, a030, w110, vb110, w030, vb030, gam, bet, out, *, relu):
  o110 = (jnp.dot(a110[...], w110[...], preferred_element_type=_f32)
          + vb110[...]) * 0.1
  o030 = (jnp.dot(a030[...], w030[...], preferred_element_type=_f32)
          + vb030[...]) * 0.1
  y = (o110 + o030) * 0.5
  m = jnp.mean(y, axis=0, keepdims=True)
  v = jnp.mean((y - m) ** 2, axis=0, keepdims=True)
  y = gam[...] * (y - m) * lax.rsqrt(v + 1e-5) + bet[...]
  if relu:
    y = jnp.maximum(y, 0.0)
  out[...] = y


def _tc1(relu):
  return pl.pallas_call(
      functools.partial(_tc1_body, relu=relu),
      out_shape=jax.ShapeDtypeStruct((N, D), _f32))


def _tc0(relu):
  return pl.pallas_call(
      functools.partial(_tc0_body, relu=relu),
      out_shape=jax.ShapeDtypeStruct((N, D), _f32))


def _pad_src(v, total, mod, shape=(NTILE, CH, 128)):
  pad = total - v.shape[0]
  fill = jnp.arange(pad, dtype=_i32) % mod
  return jnp.concatenate([v.astype(_i32), fill]).reshape(shape)


def _pad_dst(v, total):
  pad = total - v.shape[0]
  fill = N + (jnp.arange(pad, dtype=_i32) % 8)
  return jnp.concatenate([v.astype(_i32), fill]).reshape(NTILE, CH, 128)


def kernel(x0, x1, edge_index_101, edge_attr_101, edge_index_110,
           edge_attr_110, edge_index_021, edge_attr_021, edge_index_030,
           edge_attr_030, x_emb1, x_emb2, edge_emb1, edge_emb2, gin_W1,
           gin_b1, gin_W2, gin_b2, W110, b110, W021, b021, W030, b030,
           bn_gamma, bn_beta):
  R360, R18 = 28, 512
  t360 = jnp.tile((x_emb1[:, None, :] + x_emb2[None, :, :]).reshape(360, D),
                  (R360, 1))
  t18 = jnp.tile((edge_emb1[:, None, :] + edge_emb2[None, :, :]).reshape(18, D),
                 (R18, 1))
  zz = jnp.zeros((N, D), _f32)

  def spread(f, nrows, nrep):
    return f + nrows * (jnp.arange(f.shape[0], dtype=_i32) % nrep)

  f0i = _pad_src(spread(x0[:, 0] * 3 + x0[:, 1], 360, R360), NP, 360 * R360,
                 (NTILE, CI, 128))
  f1i = _pad_src(spread(x1[:, 0] * 3 + x1[:, 1], 360, R360), NP, 360 * R360,
                 (NTILE, CI, 128))

  def eidx(ei, ea):
    fa = _pad_src(spread(ea[:, 0] * 3 + ea[:, 1], 18, R18), EP, 18 * R18)
    s = _pad_src(ei[0], EP, N)
    dd = _pad_dst(ei[1], EP)
    return fa, s, dd

  fa101, s101, dd101 = eidx(edge_index_101, edge_attr_101)
  fa021, s021, dd021 = eidx(edge_index_021, edge_attr_021)
  fa110, s110, dd110 = eidx(edge_index_110, edge_attr_110)
  fa030, s030, dd030 = eidx(edge_index_030, edge_attr_030)

  h0p, h1p, p101, p021, p110, p030 = _sc_pre(
      t360, t18, zz, f0i, f1i, fa101, dd101, fa021, dd021, fa110, dd110,
      fa030, dd030)
  h0 = h0p[:N]
  h1 = h1p[:N]

  gb1 = gin_b1.reshape(1, -1)
  gb2 = gin_b2.reshape(1, -1)
  vb110 = b110.reshape(1, -1)
  vb021 = b021.reshape(1, -1)
  vb030 = b030.reshape(1, -1)

  for layer in range(2):
    a101, a021, a110, a030 = _sc_layer(
        h0, h1, p101, p021, p110, p030, s101, dd101, s021, dd021,
        s110, dd110, s030, dd030)
    gam = bn_gamma[layer].reshape(1, D)
    bet = bn_beta[layer].reshape(1, D)
    relu = layer == 0
    h1 = _tc1(relu)(h1, a101, a021, gin_W1, gb1, gin_W2, gb2, W021, vb021,
                    gam, bet)
    h0 = _tc0(relu)(a110, a030, W110, vb110, W030, vb030, gam, bet)

  return jnp.concatenate([h0, h1], axis=0)


# speedup vs baseline: 1.1289x; 1.0053x over previous
import functools

import jax
import jax.numpy as jnp
from jax import lax
from jax.experimental import pallas as pl
from jax.experimental.pallas import tpu as pltpu
from jax.experimental.pallas import tpu_sc as plsc

N = 10000
D = 128
E = 160000
NTILE = 16
CH = 80
GCH = 16
GN = CH // GCH
EP = NTILE * CH * 128
CI = 5
NP = NTILE * CI * 128
ACC_ROWS = N + 8
RPT = 624
WB = 104
TAIL = N - NTILE * RPT

_mesh = plsc.VectorSubcoreMesh(core_axis_name="c", subcore_axis_name="s")
_f32 = jnp.float32
_i32 = jnp.int32


def _make_jobs(tile, s_v, d_v, rows, acc, gsems, ssems):
  r0 = rows[0]

  def init_job(table, fidx, outp):
    def body(j, carry):
      pltpu.sync_copy(fidx.at[tile, j], s_v.at[0])
      pltpu.async_copy(table.at[s_v.at[0]], r0, gsems[0]).wait()
      pltpu.sync_copy(r0, outp.at[pl.ds(tile * CI * 128 + j * 128, 128)])
      return carry

    lax.fori_loop(0, CI, body, 0)

  def seg_job(table, s2, d2, outp, base):
    def ib(k, carry):
      off = tile * RPT + k * WB
      pltpu.sync_copy(base.at[pl.ds(off, WB)], r0.at[pl.ds(0, WB)])
      pltpu.sync_copy(r0.at[pl.ds(0, WB)], acc.at[pl.ds(off, WB)])
      return carry

    lax.fori_loop(0, RPT // WB, ib, 0)

    @pl.when(tile == 0)
    def _():
      pltpu.sync_copy(base.at[pl.ds(NTILE * RPT, TAIL)], r0.at[pl.ds(0, TAIL)])
      pltpu.sync_copy(r0.at[pl.ds(0, TAIL)], acc.at[pl.ds(NTILE * RPT, TAIL)])

    plsc.subcore_barrier()

    r1 = rows[1]

    def eb(p, carry):
      @pl.when(p > 0)
      def _():
        pltpu.make_async_copy(r1, acc.at[d_v.at[15]], ssems[1]).wait()

      @pl.when(p % 8 == 0)
      def _():
        q = p // 8
        pltpu.sync_copy(s2.at[tile, pl.ds(q * GCH, GCH)], s_v)
        pltpu.sync_copy(d2.at[tile, pl.ds(q * GCH, GCH)], d_v)

      j = (p % 8) * 2
      c0 = pltpu.async_copy(table.at[s_v.at[j]], r0, gsems[0])
      c1 = pltpu.async_copy(table.at[s_v.at[j + 1]], r1, gsems[1])
      c0.wait()
      pltpu.sync_copy(r0, acc.at[d_v.at[j]], add=True)
      c1.wait()
      pltpu.async_copy(r1, acc.at[d_v.at[j + 1]], ssems[1], add=True)
      return carry

    lax.fori_loop(0, CH // 2, eb, 0)
    pltpu.make_async_copy(r1, acc.at[d_v.at[15]], ssems[1]).wait()
    plsc.subcore_barrier()

    def ob(k, carry):
      off = tile * RPT + k * WB
      pltpu.sync_copy(acc.at[pl.ds(off, WB)], r0.at[pl.ds(0, WB)])
      pltpu.sync_copy(r0.at[pl.ds(0, WB)], outp.at[pl.ds(off, WB)])
      return carry

    lax.fori_loop(0, RPT // WB, ob, 0)

    @pl.when(tile == 0)
    def _():
      pltpu.sync_copy(acc.at[pl.ds(NTILE * RPT, TAIL)], rows[1].at[pl.ds(0, TAIL)])
      pltpu.sync_copy(rows[1].at[pl.ds(0, TAIL)],
                      outp.at[pl.ds(NTILE * RPT, TAIL)])

  return init_job, seg_job


def _fused_body(t360, t18, zz, f0i, f1i, fa101, dd101, fa021, dd021, fa110,
                dd110, fa030, dd030, s101, s021, s110, s030,
                h0p, h1p, p101, p021, p110, p030, a101, a021, a110, a030,
                s_v, d_v, r0, r1, acc, gs0, gs1, ss0, ss1):
  tile = lax.axis_index("s")
  core = lax.axis_index("c")
  init_job, seg_job = _make_jobs(tile, s_v, d_v, (r0, r1), acc,
                                 (gs0, gs1), (ss0, ss1))

  @pl.when(core == 0)
  def _():
    init_job(t360, f0i, h0p)
    plsc.subcore_barrier()
    seg_job(t18, fa021, dd021, p021, zz)
    seg_job(t18, fa030, dd030, p030, zz)
    seg_job(h0p, s021, dd021, a021, p021)
    seg_job(h0p, s030, dd030, a030, p030)

  @pl.when(core == 1)
  def _():
    init_job(t360, f1i, h1p)
    plsc.subcore_barrier()
    seg_job(t18, fa101, dd101, p101, zz)
    seg_job(t18, fa110, dd110, p110, zz)
    seg_job(h1p, s101, dd101, a101, p101)
    seg_job(h1p, s110, dd110, a110, p110)


def _layer_body(h0, h1, q101, q021, q110, q030, s101, dd101, s021, dd021,
                s110, dd110, s030, dd030, a101, a021, a110, a030,
                s_v, d_v, r0, r1, acc, gs0, gs1, ss0, ss1):
  tile = lax.axis_index("s")
  core = lax.axis_index("c")
  _, seg_job = _make_jobs(tile, s_v, d_v, (r0, r1), acc,
                          (gs0, gs1), (ss0, ss1))

  @pl.when(core == 0)
  def _():
    seg_job(h1, s101, dd101, a101, q101)
    seg_job(h0, s030, dd030, a030, q030)

  @pl.when(core == 1)
  def _():
    seg_job(h0, s021, dd021, a021, q021)
    seg_job(h1, s110, dd110, a110, q110)


_SC_SCRATCH = [
    pltpu.VMEM((GCH, 128), _i32),
    pltpu.VMEM((GCH, 128), _i32),
    pltpu.VMEM((128, D), _f32),
    pltpu.VMEM((128, D), _f32),
    pltpu.VMEM_SHARED((ACC_ROWS, D), _f32),
    pltpu.SemaphoreType.DMA,
    pltpu.SemaphoreType.DMA,
    pltpu.SemaphoreType.DMA,
    pltpu.SemaphoreType.DMA,
]

_sc_fused = pl.kernel(
    _fused_body,
    out_type=[jax.ShapeDtypeStruct((NP, D), _f32),
              jax.ShapeDtypeStruct((NP, D), _f32)] +
             [jax.ShapeDtypeStruct((N, D), _f32)] * 8,
    mesh=_mesh,
    scratch_types=_SC_SCRATCH,
)

_sc_layer = pl.kernel(
    _layer_body,
    out_type=[jax.ShapeDtypeStruct((N, D), _f32)] * 4,
    mesh=_mesh,
    scratch_types=_SC_SCRATCH,
)


def _tc1_body(h1, a101, a021, gw1, gb1, gw2, gb2, w021, vb021, gam, bet, out,
              *, relu):
  x = a101[...] + 1.1 * h1[...]
  hh = jnp.maximum(
      jnp.dot(x, gw1[...], preferred_element_type=_f32) + gb1[...], 0.0)
  hgin = jnp.dot(hh, gw2[...], preferred_element_type=_f32) + gb2[...]
  o021 = (jnp.dot(a021[...], w021[...], preferred_element_type=_f32)
          + vb021[...]) * 0.1
  y = (hgin + o021) * 0.5
  m = jnp.mean(y, axis=0, keepdims=True)
  v = jnp.mean((y - m) ** 2, axis=0, keepdims=True)
  y = gam[...] * (y - m) * lax.rsqrt(v + 1e-5) + bet[...]
  if relu:
    y = jnp.maximum(y, 0.0)
  out[...] = y


def _tc0_body(a110, a030, w110, vb110, w030, vb030, gam, bet, out, *, relu):
  o110 = (jnp.dot(a110[...], w110[...], preferred_element_type=_f32)
          + vb110[...]) * 0.1
  o030 = (jnp.dot(a030[...], w030[...], preferred_element_type=_f32)
          + vb030[...]) * 0.1
  y = (o110 + o030) * 0.5
  m = jnp.mean(y, axis=0, keepdims=True)
  v = jnp.mean((y - m) ** 2, axis=0, keepdims=True)
  y = gam[...] * (y - m) * lax.rsqrt(v + 1e-5) + bet[...]
  if relu:
    y = jnp.maximum(y, 0.0)
  out[...] = y


def _tc1(relu):
  return pl.pallas_call(
      functools.partial(_tc1_body, relu=relu),
      out_shape=jax.ShapeDtypeStruct((N, D), _f32))


def _tc0(relu):
  return pl.pallas_call(
      functools.partial(_tc0_body, relu=relu),
      out_shape=jax.ShapeDtypeStruct((N, D), _f32))


def _pad_src(v, total, mod, shape=(NTILE, CH, 128)):
  pad = total - v.shape[0]
  fill = jnp.arange(pad, dtype=_i32) % mod
  return jnp.concatenate([v.astype(_i32), fill]).reshape(shape)


def _pad_dst(v, total):
  pad = total - v.shape[0]
  fill = N + (jnp.arange(pad, dtype=_i32) % 8)
  return jnp.concatenate([v.astype(_i32), fill]).reshape(NTILE, CH, 128)


def kernel(x0, x1, edge_index_101, edge_attr_101, edge_index_110,
           edge_attr_110, edge_index_021, edge_attr_021, edge_index_030,
           edge_attr_030, x_emb1, x_emb2, edge_emb1, edge_emb2, gin_W1,
           gin_b1, gin_W2, gin_b2, W110, b110, W021, b021, W030, b030,
           bn_gamma, bn_beta):
  R360, R18 = 28, 512
  t360 = jnp.tile((x_emb1[:, None, :] + x_emb2[None, :, :]).reshape(360, D),
                  (R360, 1))
  t18 = jnp.tile((edge_emb1[:, None, :] + edge_emb2[None, :, :]).reshape(18, D),
                 (R18, 1))
  zz = jnp.zeros((N, D), _f32)

  def spread(f, nrows, nrep):
    return f + nrows * (jnp.arange(f.shape[0], dtype=_i32) % nrep)

  f0i = _pad_src(spread(x0[:, 0] * 3 + x0[:, 1], 360, R360), NP, 360 * R360,
                 (NTILE, CI, 128))
  f1i = _pad_src(spread(x1[:, 0] * 3 + x1[:, 1], 360, R360), NP, 360 * R360,
                 (NTILE, CI, 128))

  def eidx(ei, ea):
    fa = _pad_src(spread(ea[:, 0] * 3 + ea[:, 1], 18, R18), EP, 18 * R18)
    s = _pad_src(ei[0], EP, N)
    dd = _pad_dst(ei[1], EP)
    return fa, s, dd

  fa101, s101, dd101 = eidx(edge_index_101, edge_attr_101)
  fa021, s021, dd021 = eidx(edge_index_021, edge_attr_021)
  fa110, s110, dd110 = eidx(edge_index_110, edge_attr_110)
  fa030, s030, dd030 = eidx(edge_index_030, edge_attr_030)

  (h0p, h1p, p101, p021, p110, p030, a101, a021, a110, a030) = _sc_fused(
      t360, t18, zz, f0i, f1i, fa101, dd101, fa021, dd021, fa110, dd110,
      fa030, dd030, s101, s021, s110, s030)
  h0 = h0p[:N]
  h1 = h1p[:N]

  gb1 = gin_b1.reshape(1, -1)
  gb2 = gin_b2.reshape(1, -1)
  vb110 = b110.reshape(1, -1)
  vb021 = b021.reshape(1, -1)
  vb030 = b030.reshape(1, -1)

  for layer in range(2):
    if layer > 0:
      a101, a021, a110, a030 = _sc_layer(
          h0, h1, p101, p021, p110, p030, s101, dd101, s021, dd021,
          s110, dd110, s030, dd030)
    gam = bn_gamma[layer].reshape(1, D)
    bet = bn_beta[layer].reshape(1, D)
    relu = layer == 0
    h1 = _tc1(relu)(h1, a101, a021, gin_W1, gb1, gin_W2, gb2, W021, vb021,
                    gam, bet)
    h0 = _tc0(relu)(a110, a030, W110, vb110, W030, vb030, gam, bet)

  return jnp.concatenate([h0, h1], axis=0)
